# baseline (device time: 17963 ns/iter reference)
import jax
import jax.numpy as jnp
from jax import lax
from jax.experimental import pallas as pl
from jax.experimental.pallas import tpu as pltpu

N_DEV = 4


def kernel(x, w_mat):
    m_per, k = x.shape
    _, n_per = w_mat.shape
    m_half = m_per // 2
    m_q = m_per // 4

    def body(x_ref, w_ref, out_ref,
             x_vmem, w_vmem, out_vmem, mine, from_l, from_r, diag_a, diag_b,
             send_sems, recv_sems, load_sems, store_sems):
        my_pos = lax.axis_index("i")
        left = lax.rem(my_pos + N_DEV - 1, N_DEV)
        right = lax.rem(my_pos + 1, N_DEV)
        diag = lax.rem(my_pos + 2, N_DEV)

        def q(ref, i, n=1):
            return ref.at[pl.ds(i * m_q, n * m_q), :]

        x_loads = [
            pltpu.make_async_copy(q(x_ref, i), q(x_vmem, i), load_sems.at[i])
            for i in range(4)
        ]
        for i in (0, 2, 1, 3):
            x_loads[i].start()
        w_load = pltpu.make_async_copy(w_ref, w_vmem, load_sems.at[4])
        w_load.start()

        barrier_sem = pltpu.get_barrier_semaphore()
        for nbr in [left, right]:
            pl.semaphore_signal(
                barrier_sem, inc=1,
                device_id=(nbr,), device_id_type=pl.DeviceIdType.MESH,
            )

        def rcopy(src, dst, sem_idx, dev):
            return pltpu.make_async_remote_copy(
                src_ref=src, dst_ref=dst,
                send_sem=send_sems.at[sem_idx], recv_sem=recv_sems.at[sem_idx],
                device_id=(dev,), device_id_type=pl.DeviceIdType.MESH,
            )

        def send_mine(i):
            rcopy(q(mine, i), q(from_l, i), i, right).start()

        def send_mine_left(i):
            rcopy(q(mine, i), q(from_r, i), 4 + i, left).start()

        x_loads[0].wait()
        mine[pl.ds(0, m_q), :] = x_vmem[pl.ds(0, m_q), :].astype(jnp.bfloat16)
        x_loads[2].wait()
        mine[pl.ds(2 * m_q, m_q), :] = (
            x_vmem[pl.ds(2 * m_q, m_q), :].astype(jnp.bfloat16))
        pl.semaphore_wait(barrier_sem, 2)
        send_mine(0)
        send_mine_left(2)
        x_loads[1].wait()
        mine[pl.ds(m_q, m_q), :] = (
            x_vmem[pl.ds(m_q, m_q), :].astype(jnp.bfloat16))
        send_mine(1)
        x_loads[3].wait()
        mine[pl.ds(3 * m_q, m_q), :] = (
            x_vmem[pl.ds(3 * m_q, m_q), :].astype(jnp.bfloat16))
        send_mine_left(3)
        send_mine(2)
        send_mine(3)
        send_mine_left(0)
        send_mine_left(1)

        out_stores = []

        def gemm_store(src_block, origin_row, blk):
            out_vmem[pl.ds(blk * m_half, m_half), :] = jnp.dot(
                src_block, w, preferred_element_type=jnp.float32
            ).astype(jnp.bfloat16)
            st = pltpu.make_async_copy(
                out_vmem.at[pl.ds(blk * m_half, m_half), :],
                out_ref.at[pl.ds(origin_row, m_half), :],
                store_sems.at[blk])
            st.start()
            out_stores.append(st)

        w_load.wait()
        w = w_vmem[...].astype(jnp.bfloat16)
        gemm_store(mine[:m_half, :], my_pos * m_per, 0)
        gemm_store(mine[m_half:, :], my_pos * m_per + m_half, 1)

        fwds = []
        rcopy(q(from_l, 0), q(from_l, 0), 0, left).wait_recv()
        fwds.append(rcopy(q(from_l, 0), q(diag_a, 0), 8, right))
        fwds[-1].start()
        rcopy(q(from_r, 2), q(from_r, 2), 6, right).wait_recv()
        fwds.append(rcopy(q(from_r, 2), q(diag_b, 0), 10, left))
        fwds[-1].start()
        rcopy(q(from_l, 1), q(from_l, 1), 1, left).wait_recv()
        fwds.append(rcopy(q(from_l, 1), q(diag_a, 1), 9, right))
        fwds[-1].start()
        rcopy(q(from_r, 3), q(from_r, 3), 7, right).wait_recv()
        fwds.append(rcopy(q(from_r, 3), q(diag_b, 1), 11, left))
        fwds[-1].start()

        gemm_store(from_l[:m_half, :], left * m_per, 2)
        gemm_store(from_r[m_half:, :], right * m_per + m_half, 5)
        rcopy(q(from_l, 2), q(from_l, 2), 2, left).wait_recv()
        rcopy(q(from_l, 3), q(from_l, 3), 3, left).wait_recv()
        gemm_store(from_l[m_half:, :], left * m_per + m_half, 3)
        rcopy(q(from_r, 0), q(from_r, 0), 4, right).wait_recv()
        rcopy(q(from_r, 1), q(from_r, 1), 5, right).wait_recv()
        gemm_store(from_r[:m_half, :], right * m_per, 4)

        rcopy(q(diag_a, 0), q(diag_a, 0), 8, left).wait_recv()
        rcopy(q(diag_a, 1), q(diag_a, 1), 9, left).wait_recv()
        gemm_store(diag_a[...], diag * m_per, 6)
        rcopy(q(diag_b, 0), q(diag_b, 0), 10, right).wait_recv()
        rcopy(q(diag_b, 1), q(diag_b, 1), 11, right).wait_recv()
        gemm_store(diag_b[...], diag * m_per + m_half, 7)

        for i in range(8):
            pltpu.make_async_remote_copy(
                src_ref=q(mine, i % 4), dst_ref=q(mine, i % 4),
                send_sem=send_sems.at[i], recv_sem=recv_sems.at[i],
                device_id=(right,), device_id_type=pl.DeviceIdType.MESH,
            ).wait_send()
        for f in fwds:
            f.wait_send()
        for st in out_stores:
            st.wait()

    out_shape = jax.ShapeDtypeStruct((N_DEV * m_per, n_per), jnp.bfloat16)
    return pl.pallas_call(
        body,
        out_shape=out_shape,
        in_specs=[
            pl.BlockSpec(memory_space=pl.ANY),
            pl.BlockSpec(memory_space=pl.ANY),
        ],
        out_specs=pl.BlockSpec(memory_space=pl.ANY),
        scratch_shapes=[
            pltpu.VMEM((m_per, k), jnp.float32),
            pltpu.VMEM((k, n_per), jnp.float32),
            pltpu.VMEM((N_DEV * m_per, n_per), jnp.bfloat16),
            pltpu.VMEM((m_per, k), jnp.bfloat16),
            pltpu.VMEM((m_per, k), jnp.bfloat16),
            pltpu.VMEM((m_per, k), jnp.bfloat16),
            pltpu.VMEM((m_half, k), jnp.bfloat16),
            pltpu.VMEM((m_half, k), jnp.bfloat16),
            pltpu.SemaphoreType.DMA((12,)),
            pltpu.SemaphoreType.DMA((12,)),
            pltpu.SemaphoreType.DMA((5,)),
            pltpu.SemaphoreType.DMA((8,)),
        ],
        compiler_params=pltpu.CompilerParams(collective_id=0),
    )(x, w_mat)


# device time: 17632 ns/iter; 1.0188x vs baseline; 1.0188x over previous
import jax
import jax.numpy as jnp
from jax import lax
from jax.experimental import pallas as pl
from jax.experimental.pallas import tpu as pltpu

N_DEV = 4


def kernel(x, w_mat):
    m_per, k = x.shape
    _, n_per = w_mat.shape
    m_half = m_per // 2

    def body(x_ref, w_ref, out_ref,
             out_vmem, mine, from_l, from_r, diag_a, diag_b,
             send_sems, recv_sems, store_sems):
        my_pos = lax.axis_index("i")
        left = lax.rem(my_pos + N_DEV - 1, N_DEV)
        right = lax.rem(my_pos + 1, N_DEV)
        diag = lax.rem(my_pos + 2, N_DEV)

        barrier_sem = pltpu.get_barrier_semaphore()
        for nbr in [left, right]:
            pl.semaphore_signal(
                barrier_sem, inc=1,
                device_id=(nbr,), device_id_type=pl.DeviceIdType.MESH,
            )

        mine[0] = x_ref[:m_half, :].astype(jnp.bfloat16)
        mine[1] = x_ref[m_half:, :].astype(jnp.bfloat16)
        pl.semaphore_wait(barrier_sem, 2)

        def rcopy(src, dst, sem_idx, dev):
            return pltpu.make_async_remote_copy(
                src_ref=src, dst_ref=dst,
                send_sem=send_sems.at[sem_idx], recv_sem=recv_sems.at[sem_idx],
                device_id=(dev,), device_id_type=pl.DeviceIdType.MESH,
            )

        sends = [
            rcopy(mine.at[0], from_l.at[0], 0, right),
            rcopy(mine.at[1], from_r.at[1], 1, left),
            rcopy(mine.at[1], from_l.at[1], 2, right),
            rcopy(mine.at[0], from_r.at[0], 3, left),
        ]
        for s in sends:
            s.start()

        out_stores = []

        def gemm_store(src_block, origin_row, blk):
            out_vmem[pl.ds(blk * m_half, m_half), :] = jnp.dot(
                src_block, w, preferred_element_type=jnp.float32
            ).astype(jnp.bfloat16)
            st = pltpu.make_async_copy(
                out_vmem.at[pl.ds(blk * m_half, m_half), :],
                out_ref.at[pl.ds(origin_row, m_half), :],
                store_sems.at[blk])
            st.start()
            out_stores.append(st)

        w = w_ref[...].astype(jnp.bfloat16)
        gemm_store(mine[0], my_pos * m_per, 0)
        gemm_store(mine[1], my_pos * m_per + m_half, 1)

        rcopy(from_l.at[0], from_l.at[0], 0, left).wait_recv()
        fwd_r = rcopy(from_l.at[0], diag_a, 4, right)
        fwd_r.start()
        rcopy(from_r.at[1], from_r.at[1], 1, right).wait_recv()
        fwd_l = rcopy(from_r.at[1], diag_b, 5, left)
        fwd_l.start()

        rcopy(from_l.at[1], from_l.at[1], 2, left).wait_recv()
        gemm_store(from_l[0], left * m_per, 2)
        gemm_store(from_l[1], left * m_per + m_half, 3)

        rcopy(from_r.at[0], from_r.at[0], 3, right).wait_recv()
        gemm_store(from_r[0], right * m_per, 4)
        gemm_store(from_r[1], right * m_per + m_half, 5)

        rcopy(diag_a, diag_a, 4, left).wait_recv()
        gemm_store(diag_a[...], diag * m_per, 6)
        rcopy(diag_b, diag_b, 5, right).wait_recv()
        gemm_store(diag_b[...], diag * m_per + m_half, 7)

        for s in sends:
            s.wait_send()
        fwd_r.wait_send()
        fwd_l.wait_send()
        for st in out_stores:
            st.wait()

    out_shape = jax.ShapeDtypeStruct((N_DEV * m_per, n_per), jnp.bfloat16)
    return pl.pallas_call(
        body,
        out_shape=out_shape,
        in_specs=[
            pl.BlockSpec(memory_space=pltpu.VMEM),
            pl.BlockSpec(memory_space=pltpu.VMEM),
        ],
        out_specs=pl.BlockSpec(memory_space=pl.ANY),
        scratch_shapes=[
            pltpu.VMEM((N_DEV * m_per, n_per), jnp.bfloat16),
            pltpu.VMEM((2, m_half, k), jnp.bfloat16),
            pltpu.VMEM((2, m_half, k), jnp.bfloat16),
            pltpu.VMEM((2, m_half, k), jnp.bfloat16),
            pltpu.VMEM((m_half, k), jnp.bfloat16),
            pltpu.VMEM((m_half, k), jnp.bfloat16),
            pltpu.SemaphoreType.DMA((6,)),
            pltpu.SemaphoreType.DMA((6,)),
            pltpu.SemaphoreType.DMA((8,)),
        ],
        compiler_params=pltpu.CompilerParams(collective_id=0),
    )(x, w_mat)
